# Initial kernel scaffold; baseline (speedup 1.0000x reference)
import jax
import jax.numpy as jnp
from jax.experimental import pallas as pl


def kernel(x, edge_index, Wl1, Wr1, b1, Wl2, Wr2, b2, Wl3, Wr3, b3, g1, be1, g2, be2, g3, be3, Wla, Wra, ba, Wls, Wrs, bs, Wle, Wre, be):
    raise NotImplementedError("write your pallas kernel here")


# speedup vs baseline: 8.7249x; 8.7249x over previous
import functools

import jax
import jax.numpy as jnp
from jax import lax
from jax.experimental import pallas as pl
from jax.experimental.pallas import tpu as pltpu
from jax.experimental.pallas import tpu_sc as plsc

N = 10000
E = 320000
D = 128
OUT_CAT = 28

NC = 2
NS = 16
NW = NC * NS
EPT = E // NW
CHUNK = 80
NCHUNK = EPT // CHUNK
G = 25
NG = NCHUNK // G
GP = (G - 1) // 2
NZT = 10
RPT = N // NZT
ZR = 40
NZC = RPT // ZR


@functools.cache
def _sc_mesh():
    return plsc.VectorSubcoreMesh(
        core_axis_name="c", subcore_axis_name="s",
        num_cores=NC, num_subcores=NS)


@functools.cache
def _sc_agg_kernel():
  @functools.partial(
      pl.kernel,
      out_type=jax.ShapeDtypeStruct((NC, N, D), jnp.float32),
      mesh=_sc_mesh(),
      scratch_types=[
          pltpu.VMEM((G, CHUNK), jnp.int32),
          pltpu.VMEM((G, CHUNK), jnp.int32),
          pltpu.VMEM((2, CHUNK, D), jnp.float32),
          pltpu.VMEM((ZR, D), jnp.float32),
          pltpu.VMEM_SHARED((N, D), jnp.float32),
          pltpu.SemaphoreType.DMA,
      ],
  )
  def _sc_agg(h_hbm, src_hbm, dst_hbm, zc, out_hbm,
              srcg, dstg, rows, zbuf, acc, sem):
    c = lax.axis_index("c")
    s = lax.axis_index("s")
    wid = c * NS + s

    pltpu.sync_copy(zc, zbuf)

    @pl.when(s < NZT)
    def _():
        def zoff(i):
            return pl.multiple_of(s * RPT + i * ZR, 8)

        def zfire(i, carry):
            pltpu.async_copy(zbuf, acc.at[pl.ds(zoff(i), ZR)], sem)
            return carry

        def zdrain(i, carry):
            pltpu.make_async_copy(zbuf, acc.at[pl.ds(zoff(i), ZR)],
                                  sem).wait()
            return carry

        lax.fori_loop(0, NZC, zfire, 0)
        lax.fori_loop(0, NZC, zdrain, 0)

    plsc.subcore_barrier()

    def gather(b, j):
        pltpu.async_copy(h_hbm.at[srcg.at[j]], rows.at[b], sem)

    def wait(b, j):
        pltpu.make_async_copy(h_hbm.at[srcg.at[j]], rows.at[b], sem).wait()

    def scatter(b, j):
        pltpu.sync_copy(rows.at[b], acc.at[dstg.at[j]], add=True)

    for g in range(NG):
        pltpu.sync_copy(src_hbm.at[wid, g], srcg)
        pltpu.sync_copy(dst_hbm.at[wid, g], dstg)
        gather(0, 0)

        def pair(t, carry):
            j0 = 2 * t
            gather(1, j0 + 1)
            wait(0, j0)
            scatter(0, j0)
            gather(0, j0 + 2)
            wait(1, j0 + 1)
            scatter(1, j0 + 1)
            return carry

        lax.fori_loop(0, GP, pair, 0)
        wait(0, G - 1)
        scatter(0, G - 1)

    plsc.subcore_barrier()

    @pl.when(s < NZT)
    def _():
        off = pl.multiple_of(s * RPT, 8)
        pltpu.sync_copy(acc.at[pl.ds(off, RPT)],
                        out_hbm.at[c, pl.ds(off, RPT)])

  return _sc_agg


@functools.cache
def _sc_deg_kernel():
  @functools.partial(
      pl.kernel,
      out_type=jax.ShapeDtypeStruct((NC, N, D), jnp.float32),
      mesh=_sc_mesh(),
      scratch_types=[
          pltpu.VMEM((G, CHUNK), jnp.int32),
          pltpu.VMEM((ZR, D), jnp.float32),
          pltpu.VMEM((CHUNK, D), jnp.float32),
          pltpu.VMEM_SHARED((N, D), jnp.float32),
          pltpu.SemaphoreType.DMA,
      ],
  )
  def _sc_deg(dst_hbm, zc, onesc, deg_hbm, dstg, zbuf, ones, dacc, sem):
    c = lax.axis_index("c")
    s = lax.axis_index("s")
    wid = c * NS + s

    pltpu.sync_copy(zc, zbuf)
    pltpu.sync_copy(onesc, ones)

    @pl.when(s < NZT)
    def _():
        def zoff(i):
            return pl.multiple_of(s * RPT + i * ZR, 8)

        def zfire(i, carry):
            pltpu.async_copy(zbuf, dacc.at[pl.ds(zoff(i), ZR)], sem)
            return carry

        def zdrain(i, carry):
            pltpu.make_async_copy(zbuf, dacc.at[pl.ds(zoff(i), ZR)],
                                  sem).wait()
            return carry

        lax.fori_loop(0, NZC, zfire, 0)
        lax.fori_loop(0, NZC, zdrain, 0)

    plsc.subcore_barrier()

    for g in range(NG):
        pltpu.sync_copy(dst_hbm.at[wid, g], dstg)

        def step(j, carry):
            pltpu.sync_copy(ones, dacc.at[dstg.at[j]], add=True)
            return carry

        lax.fori_loop(0, G, step, 0)
    plsc.subcore_barrier()

    @pl.when(s < NZT)
    def _():
        off = pl.multiple_of(s * RPT, 8)
        pltpu.sync_copy(dacc.at[pl.ds(off, RPT)],
                        deg_hbm.at[c, pl.ds(off, RPT)])

  return _sc_deg


RB = 1000
NB = N // RB


def _layer1_tc(p_ref, degp_ref, h_ref, wl_ref, wr_ref, b_ref,
               z_ref, stats_ref, inv_ref):
    i = pl.program_id(0)
    deg = degp_ref[0, :, 0] + degp_ref[1, :, 0]
    inv = 1.0 / jnp.maximum(deg, 1.0)
    inv_ref[...] = inv[:, None]
    mean = (p_ref[0] + p_ref[1]) * inv[:, None]
    z = (jnp.dot(mean, wl_ref[...], preferred_element_type=jnp.float32)
         + jnp.dot(h_ref[...], wr_ref[...], preferred_element_type=jnp.float32)
         + b_ref[...])
    z_ref[...] = z

    @pl.when(i == 0)
    def _():
        stats_ref[...] = jnp.zeros_like(stats_ref)

    stats_ref[0:1, :] = stats_ref[0:1, :] + jnp.sum(z, axis=0, keepdims=True)
    stats_ref[1:2, :] = stats_ref[1:2, :] + jnp.sum(z * z, axis=0,
                                                    keepdims=True)


def _layerA_tc(p_ref, inv_ref, h_ref, wl_ref, wr_ref, b_ref, z_ref, stats_ref):
    i = pl.program_id(0)
    mean = (p_ref[0] + p_ref[1]) * inv_ref[...]
    z = (jnp.dot(mean, wl_ref[...], preferred_element_type=jnp.float32)
         + jnp.dot(h_ref[...], wr_ref[...], preferred_element_type=jnp.float32)
         + b_ref[...])
    z_ref[...] = z

    @pl.when(i == 0)
    def _():
        stats_ref[...] = jnp.zeros_like(stats_ref)

    stats_ref[0:1, :] = stats_ref[0:1, :] + jnp.sum(z, axis=0, keepdims=True)
    stats_ref[1:2, :] = stats_ref[1:2, :] + jnp.sum(z * z, axis=0,
                                                    keepdims=True)


def _bn_relu_tc(z_ref, stats_ref, g_ref, be_ref, o_ref):
    m = stats_ref[0:1, :] * (1.0 / N)
    v = stats_ref[1:2, :] * (1.0 / N) - m * m
    o_ref[...] = jnp.maximum(
        (z_ref[...] - m) * lax.rsqrt(v + 1e-5) * g_ref[...] + be_ref[...], 0.0)


def _heads_tc(p_ref, inv_ref, h_ref, wl_ref, wr_ref, b_ref, o_ref):
    mean = (p_ref[0] + p_ref[1]) * inv_ref[...]
    o_ref[...] = (jnp.dot(mean, wl_ref[...], preferred_element_type=jnp.float32)
                  + jnp.dot(h_ref[...], wr_ref[...],
                            preferred_element_type=jnp.float32)
                  + b_ref[...])


def _call_layer1(p, degp, h, wlT, wrT, b):
    return pl.pallas_call(
        _layer1_tc,
        grid=(NB,),
        in_specs=[
            pl.BlockSpec((2, RB, D), lambda i: (0, i, 0)),
            pl.BlockSpec((2, RB, D), lambda i: (0, i, 0)),
            pl.BlockSpec((RB, D), lambda i: (i, 0)),
            pl.BlockSpec((D, D), lambda i: (0, 0)),
            pl.BlockSpec((D, D), lambda i: (0, 0)),
            pl.BlockSpec((1, D), lambda i: (0, 0)),
        ],
        out_specs=[
            pl.BlockSpec((RB, D), lambda i: (i, 0)),
            pl.BlockSpec((2, D), lambda i: (0, 0)),
            pl.BlockSpec((RB, 1), lambda i: (i, 0)),
        ],
        out_shape=[
            jax.ShapeDtypeStruct((N, D), jnp.float32),
            jax.ShapeDtypeStruct((2, D), jnp.float32),
            jax.ShapeDtypeStruct((N, 1), jnp.float32),
        ],
    )(p, degp, h, wlT, wrT, b)


def _call_layerA(p, inv, h, wlT, wrT, b):
    return pl.pallas_call(
        _layerA_tc,
        grid=(NB,),
        in_specs=[
            pl.BlockSpec((2, RB, D), lambda i: (0, i, 0)),
            pl.BlockSpec((RB, 1), lambda i: (i, 0)),
            pl.BlockSpec((RB, D), lambda i: (i, 0)),
            pl.BlockSpec((D, D), lambda i: (0, 0)),
            pl.BlockSpec((D, D), lambda i: (0, 0)),
            pl.BlockSpec((1, D), lambda i: (0, 0)),
        ],
        out_specs=[
            pl.BlockSpec((RB, D), lambda i: (i, 0)),
            pl.BlockSpec((2, D), lambda i: (0, 0)),
        ],
        out_shape=[
            jax.ShapeDtypeStruct((N, D), jnp.float32),
            jax.ShapeDtypeStruct((2, D), jnp.float32),
        ],
    )(p, inv, h, wlT, wrT, b)


def _call_bn(z, stats, g, be):
    return pl.pallas_call(
        _bn_relu_tc,
        grid=(NB,),
        in_specs=[
            pl.BlockSpec((RB, D), lambda i: (i, 0)),
            pl.BlockSpec((2, D), lambda i: (0, 0)),
            pl.BlockSpec((1, D), lambda i: (0, 0)),
            pl.BlockSpec((1, D), lambda i: (0, 0)),
        ],
        out_specs=pl.BlockSpec((RB, D), lambda i: (i, 0)),
        out_shape=jax.ShapeDtypeStruct((N, D), jnp.float32),
    )(z, stats, g, be)


def _call_heads(p, inv, h, wlT, wrT, b):
    return pl.pallas_call(
        _heads_tc,
        grid=(NB,),
        in_specs=[
            pl.BlockSpec((2, RB, D), lambda i: (0, i, 0)),
            pl.BlockSpec((RB, 1), lambda i: (i, 0)),
            pl.BlockSpec((RB, D), lambda i: (i, 0)),
            pl.BlockSpec((D, OUT_CAT), lambda i: (0, 0)),
            pl.BlockSpec((D, OUT_CAT), lambda i: (0, 0)),
            pl.BlockSpec((1, OUT_CAT), lambda i: (0, 0)),
        ],
        out_specs=pl.BlockSpec((RB, OUT_CAT), lambda i: (i, 0)),
        out_shape=jax.ShapeDtypeStruct((N, OUT_CAT), jnp.float32),
    )(p, inv, h, wlT, wrT, b)


def kernel(x, edge_index, Wl1, Wr1, b1, Wl2, Wr2, b2, Wl3, Wr3, b3,
           g1, be1, g2, be2, g3, be3,
           Wla, Wra, ba, Wls, Wrs, bs, Wle, Wre, be):
    src1 = edge_index[0].reshape(NW, NG, G, CHUNK)
    dst1 = edge_index[1].reshape(NW, NG, G, CHUNK)
    zc = jnp.zeros((ZR, D), jnp.float32)
    onesc = jnp.ones((CHUNK, D), jnp.float32)

    sc_agg = _sc_agg_kernel()
    degp = _sc_deg_kernel()(dst1, zc, onesc)
    p1 = sc_agg(x, src1, dst1, zc)
    z1, stats1, inv = _call_layer1(p1, degp, x, Wl1.T, Wr1.T, b1.reshape(1, D))
    h1 = _call_bn(z1, stats1, g1.reshape(1, D), be1.reshape(1, D))

    p2 = sc_agg(h1, src1, dst1, zc)
    z2, stats2 = _call_layerA(p2, inv, h1, Wl2.T, Wr2.T, b2.reshape(1, D))
    h2 = _call_bn(z2, stats2, g2.reshape(1, D), be2.reshape(1, D))

    p3 = sc_agg(h2, src1, dst1, zc)
    z3, stats3 = _call_layerA(p3, inv, h2, Wl3.T, Wr3.T, b3.reshape(1, D))
    h3 = _call_bn(z3, stats3, g3.reshape(1, D), be3.reshape(1, D))

    p4 = sc_agg(h3, src1, dst1, zc)
    wlcat = jnp.concatenate([Wla, Wls, Wle], axis=0).T
    wrcat = jnp.concatenate([Wra, Wrs, Wre], axis=0).T
    bcat = jnp.concatenate([ba, bs, be], axis=0).reshape(1, OUT_CAT)
    out = _call_heads(p4, inv, h3, wlcat, wrcat, bcat)
    return out[:, :21], out[:, 21:23], out[:, 23:28]


# speedup vs baseline: 10.1461x; 1.1629x over previous
import functools

import jax
import jax.numpy as jnp
from jax import lax
from jax.experimental import pallas as pl
from jax.experimental.pallas import tpu as pltpu
from jax.experimental.pallas import tpu_sc as plsc

N = 10000
E = 320000
D = 128
OUT_CAT = 28

NC = 2
NS = 16
NW = NC * NS
EPT = E // NW
CHUNK = 80
NCHUNK = EPT // CHUNK
G = 25
NG = NCHUNK // G
GP = (G - 1) // 2
NZT = 10
RPT = N // NZT
ZR = 8
NZC = RPT // ZR
NBUF = 3
KAHEAD = 2
SLAG = 1


@functools.cache
def _sc_mesh():
    return plsc.VectorSubcoreMesh(
        core_axis_name="c", subcore_axis_name="s",
        num_cores=NC, num_subcores=NS)


@functools.cache
def _sc_agg_kernel():
  @functools.partial(
      pl.kernel,
      out_type=jax.ShapeDtypeStruct((NC, N, D), jnp.float32),
      mesh=_sc_mesh(),
      scratch_types=[
          pltpu.VMEM((2, G, CHUNK), jnp.int32),
          pltpu.VMEM((2, G, CHUNK), jnp.int32),
          pltpu.VMEM((NBUF, CHUNK, D), jnp.float32),
          pltpu.VMEM((ZR, D), jnp.float32),
          pltpu.VMEM_SHARED((N, D), jnp.float32),
          pltpu.SemaphoreType.DMA,
          pltpu.SemaphoreType.DMA,
          pltpu.SemaphoreType.DMA,
      ],
  )
  def _sc_agg(h_hbm, src_hbm, dst_hbm, zc, out_hbm,
              srcg, dstg, rows, zbuf, acc, sem, ssem, isem):
    c = lax.axis_index("c")
    s = lax.axis_index("s")
    wid = c * NS + s

    pltpu.sync_copy(zc, zbuf)

    @pl.when(s < NZT)
    def _():
        def zoff(i):
            return pl.multiple_of(s * RPT + i * ZR, 8)

        def zfire(i, carry):
            pltpu.async_copy(zbuf, acc.at[pl.ds(zoff(i), ZR)], sem)
            return carry

        def zdrain(i, carry):
            pltpu.make_async_copy(zbuf, acc.at[pl.ds(zoff(i), ZR)],
                                  sem).wait()
            return carry

        lax.fori_loop(0, NZC, zfire, 0)
        lax.fori_loop(0, NZC, zdrain, 0)

    plsc.subcore_barrier()

    pltpu.async_copy(src_hbm.at[wid, 0], srcg.at[0], isem)
    pltpu.async_copy(dst_hbm.at[wid, 0], dstg.at[0], isem)

    def group_body(g, carry):
        p = g % 2
        pltpu.make_async_copy(src_hbm.at[wid, g], srcg.at[p], isem).wait()
        pltpu.make_async_copy(dst_hbm.at[wid, g], dstg.at[p], isem).wait()

        @pl.when(g + 1 < NG)
        def _():
            pltpu.async_copy(src_hbm.at[wid, g + 1], srcg.at[1 - p], isem)
            pltpu.async_copy(dst_hbm.at[wid, g + 1], dstg.at[1 - p], isem)

        def gather(b, j):
            pltpu.async_copy(h_hbm.at[srcg.at[p, j]], rows.at[b], sem)

        def gwait(b, j):
            pltpu.make_async_copy(h_hbm.at[srcg.at[p, j]], rows.at[b],
                                  sem).wait()

        def scatter(b, j):
            pltpu.async_copy(rows.at[b], acc.at[dstg.at[p, j]], ssem, add=True)

        def swait(b, j):
            pltpu.make_async_copy(rows.at[b], acc.at[dstg.at[p, j]],
                                  ssem).wait()

        for j in range(KAHEAD):
            gather(j, j)
        for j in range(G):
            b = j % NBUF
            gwait(b, j)
            scatter(b, j)
            if j >= SLAG:
                swait((j - SLAG) % NBUF, j - SLAG)
            if j + KAHEAD < G:
                gather((j + KAHEAD) % NBUF, j + KAHEAD)
        for j in range(G - SLAG, G):
            swait(j % NBUF, j)
        return carry

    lax.fori_loop(0, NG, group_body, 0)

    plsc.subcore_barrier()

    @pl.when(s < NZT)
    def _():
        off = pl.multiple_of(s * RPT, 8)
        pltpu.sync_copy(acc.at[pl.ds(off, RPT)],
                        out_hbm.at[c, pl.ds(off, RPT)])

  return _sc_agg


@functools.cache
def _sc_deg_kernel():
  @functools.partial(
      pl.kernel,
      out_type=jax.ShapeDtypeStruct((NC, N, D), jnp.float32),
      mesh=_sc_mesh(),
      scratch_types=[
          pltpu.VMEM((2, G, CHUNK), jnp.int32),
          pltpu.VMEM((ZR, D), jnp.float32),
          pltpu.VMEM((CHUNK, D), jnp.float32),
          pltpu.VMEM_SHARED((N, D), jnp.float32),
          pltpu.SemaphoreType.DMA,
          pltpu.SemaphoreType.DMA,
      ],
  )
  def _sc_deg(dst_hbm, zc, onesc, deg_hbm, dstg, zbuf, ones, dacc, sem, isem):
    c = lax.axis_index("c")
    s = lax.axis_index("s")
    wid = c * NS + s

    pltpu.sync_copy(zc, zbuf)
    pltpu.sync_copy(onesc, ones)

    @pl.when(s < NZT)
    def _():
        def zoff(i):
            return pl.multiple_of(s * RPT + i * ZR, 8)

        def zfire(i, carry):
            pltpu.async_copy(zbuf, dacc.at[pl.ds(zoff(i), ZR)], sem)
            return carry

        def zdrain(i, carry):
            pltpu.make_async_copy(zbuf, dacc.at[pl.ds(zoff(i), ZR)],
                                  sem).wait()
            return carry

        lax.fori_loop(0, NZC, zfire, 0)
        lax.fori_loop(0, NZC, zdrain, 0)

    plsc.subcore_barrier()

    pltpu.async_copy(dst_hbm.at[wid, 0], dstg.at[0], isem)

    def dgroup_body(g, carry):
        p = g % 2
        pltpu.make_async_copy(dst_hbm.at[wid, g], dstg.at[p], isem).wait()

        @pl.when(g + 1 < NG)
        def _():
            pltpu.async_copy(dst_hbm.at[wid, g + 1], dstg.at[1 - p], isem)

        def fire(j, carry):
            pltpu.async_copy(ones, dacc.at[dstg.at[p, j]], sem, add=True)
            return carry

        def drain(j, carry):
            pltpu.make_async_copy(ones, dacc.at[dstg.at[p, j]], sem).wait()
            return carry

        lax.fori_loop(0, G, fire, 0)
        lax.fori_loop(0, G, drain, 0)
        return carry

    lax.fori_loop(0, NG, dgroup_body, 0)
    plsc.subcore_barrier()

    @pl.when(s < NZT)
    def _():
        off = pl.multiple_of(s * RPT, 8)
        pltpu.sync_copy(dacc.at[pl.ds(off, RPT)],
                        deg_hbm.at[c, pl.ds(off, RPT)])

  return _sc_deg


RB = 1000
NB = N // RB


def _layer1_tc(p_ref, degp_ref, h_ref, wl_ref, wr_ref, b_ref,
               z_ref, stats_ref, inv_ref):
    i = pl.program_id(0)
    deg = degp_ref[0, :, 0] + degp_ref[1, :, 0]
    inv = 1.0 / jnp.maximum(deg, 1.0)
    inv_ref[...] = inv[:, None]
    mean = (p_ref[0] + p_ref[1]) * inv[:, None]
    z = (jnp.dot(mean, wl_ref[...], preferred_element_type=jnp.float32)
         + jnp.dot(h_ref[...], wr_ref[...], preferred_element_type=jnp.float32)
         + b_ref[...])
    z_ref[...] = z

    @pl.when(i == 0)
    def _():
        stats_ref[...] = jnp.zeros_like(stats_ref)

    stats_ref[0:1, :] = stats_ref[0:1, :] + jnp.sum(z, axis=0, keepdims=True)
    stats_ref[1:2, :] = stats_ref[1:2, :] + jnp.sum(z * z, axis=0,
                                                    keepdims=True)


def _layerA_tc(p_ref, inv_ref, h_ref, wl_ref, wr_ref, b_ref, z_ref, stats_ref):
    i = pl.program_id(0)
    mean = (p_ref[0] + p_ref[1]) * inv_ref[...]
    z = (jnp.dot(mean, wl_ref[...], preferred_element_type=jnp.float32)
         + jnp.dot(h_ref[...], wr_ref[...], preferred_element_type=jnp.float32)
         + b_ref[...])
    z_ref[...] = z

    @pl.when(i == 0)
    def _():
        stats_ref[...] = jnp.zeros_like(stats_ref)

    stats_ref[0:1, :] = stats_ref[0:1, :] + jnp.sum(z, axis=0, keepdims=True)
    stats_ref[1:2, :] = stats_ref[1:2, :] + jnp.sum(z * z, axis=0,
                                                    keepdims=True)


def _bn_relu_tc(z_ref, stats_ref, g_ref, be_ref, o_ref):
    m = stats_ref[0:1, :] * (1.0 / N)
    v = stats_ref[1:2, :] * (1.0 / N) - m * m
    o_ref[...] = jnp.maximum(
        (z_ref[...] - m) * lax.rsqrt(v + 1e-5) * g_ref[...] + be_ref[...], 0.0)


def _heads_tc(p_ref, inv_ref, h_ref, wl_ref, wr_ref, b_ref, o_ref):
    mean = (p_ref[0] + p_ref[1]) * inv_ref[...]
    o_ref[...] = (jnp.dot(mean, wl_ref[...], preferred_element_type=jnp.float32)
                  + jnp.dot(h_ref[...], wr_ref[...],
                            preferred_element_type=jnp.float32)
                  + b_ref[...])


def _call_layer1(p, degp, h, wlT, wrT, b):
    return pl.pallas_call(
        _layer1_tc,
        grid=(NB,),
        in_specs=[
            pl.BlockSpec((2, RB, D), lambda i: (0, i, 0)),
            pl.BlockSpec((2, RB, D), lambda i: (0, i, 0)),
            pl.BlockSpec((RB, D), lambda i: (i, 0)),
            pl.BlockSpec((D, D), lambda i: (0, 0)),
            pl.BlockSpec((D, D), lambda i: (0, 0)),
            pl.BlockSpec((1, D), lambda i: (0, 0)),
        ],
        out_specs=[
            pl.BlockSpec((RB, D), lambda i: (i, 0)),
            pl.BlockSpec((2, D), lambda i: (0, 0)),
            pl.BlockSpec((RB, 1), lambda i: (i, 0)),
        ],
        out_shape=[
            jax.ShapeDtypeStruct((N, D), jnp.float32),
            jax.ShapeDtypeStruct((2, D), jnp.float32),
            jax.ShapeDtypeStruct((N, 1), jnp.float32),
        ],
    )(p, degp, h, wlT, wrT, b)


def _call_layerA(p, inv, h, wlT, wrT, b):
    return pl.pallas_call(
        _layerA_tc,
        grid=(NB,),
        in_specs=[
            pl.BlockSpec((2, RB, D), lambda i: (0, i, 0)),
            pl.BlockSpec((RB, 1), lambda i: (i, 0)),
            pl.BlockSpec((RB, D), lambda i: (i, 0)),
            pl.BlockSpec((D, D), lambda i: (0, 0)),
            pl.BlockSpec((D, D), lambda i: (0, 0)),
            pl.BlockSpec((1, D), lambda i: (0, 0)),
        ],
        out_specs=[
            pl.BlockSpec((RB, D), lambda i: (i, 0)),
            pl.BlockSpec((2, D), lambda i: (0, 0)),
        ],
        out_shape=[
            jax.ShapeDtypeStruct((N, D), jnp.float32),
            jax.ShapeDtypeStruct((2, D), jnp.float32),
        ],
    )(p, inv, h, wlT, wrT, b)


def _call_bn(z, stats, g, be):
    return pl.pallas_call(
        _bn_relu_tc,
        grid=(NB,),
        in_specs=[
            pl.BlockSpec((RB, D), lambda i: (i, 0)),
            pl.BlockSpec((2, D), lambda i: (0, 0)),
            pl.BlockSpec((1, D), lambda i: (0, 0)),
            pl.BlockSpec((1, D), lambda i: (0, 0)),
        ],
        out_specs=pl.BlockSpec((RB, D), lambda i: (i, 0)),
        out_shape=jax.ShapeDtypeStruct((N, D), jnp.float32),
    )(z, stats, g, be)


def _call_heads(p, inv, h, wlT, wrT, b):
    return pl.pallas_call(
        _heads_tc,
        grid=(NB,),
        in_specs=[
            pl.BlockSpec((2, RB, D), lambda i: (0, i, 0)),
            pl.BlockSpec((RB, 1), lambda i: (i, 0)),
            pl.BlockSpec((RB, D), lambda i: (i, 0)),
            pl.BlockSpec((D, OUT_CAT), lambda i: (0, 0)),
            pl.BlockSpec((D, OUT_CAT), lambda i: (0, 0)),
            pl.BlockSpec((1, OUT_CAT), lambda i: (0, 0)),
        ],
        out_specs=pl.BlockSpec((RB, OUT_CAT), lambda i: (i, 0)),
        out_shape=jax.ShapeDtypeStruct((N, OUT_CAT), jnp.float32),
    )(p, inv, h, wlT, wrT, b)


def kernel(x, edge_index, Wl1, Wr1, b1, Wl2, Wr2, b2, Wl3, Wr3, b3,
           g1, be1, g2, be2, g3, be3,
           Wla, Wra, ba, Wls, Wrs, bs, Wle, Wre, be):
    src1 = edge_index[0].reshape(NW, NG, G, CHUNK)
    dst1 = edge_index[1].reshape(NW, NG, G, CHUNK)
    zc = jnp.zeros((ZR, D), jnp.float32)
    onesc = jnp.ones((CHUNK, D), jnp.float32)

    sc_agg = _sc_agg_kernel()
    degp = _sc_deg_kernel()(dst1, zc, onesc)
    p1 = sc_agg(x, src1, dst1, zc)
    z1, stats1, inv = _call_layer1(p1, degp, x, Wl1.T, Wr1.T, b1.reshape(1, D))
    h1 = _call_bn(z1, stats1, g1.reshape(1, D), be1.reshape(1, D))

    p2 = sc_agg(h1, src1, dst1, zc)
    z2, stats2 = _call_layerA(p2, inv, h1, Wl2.T, Wr2.T, b2.reshape(1, D))
    h2 = _call_bn(z2, stats2, g2.reshape(1, D), be2.reshape(1, D))

    p3 = sc_agg(h2, src1, dst1, zc)
    z3, stats3 = _call_layerA(p3, inv, h2, Wl3.T, Wr3.T, b3.reshape(1, D))
    h3 = _call_bn(z3, stats3, g3.reshape(1, D), be3.reshape(1, D))

    p4 = sc_agg(h3, src1, dst1, zc)
    wlcat = jnp.concatenate([Wla, Wls, Wle], axis=0).T
    wrcat = jnp.concatenate([Wra, Wrs, Wre], axis=0).T
    bcat = jnp.concatenate([ba, bs, be], axis=0).reshape(1, OUT_CAT)
    out = _call_heads(p4, inv, h3, wlcat, wrcat, bcat)
    return out[:, :21], out[:, 21:23], out[:, 23:28]
